# baseline (speedup 1.0000x reference)
import functools

import jax
import jax.numpy as jnp
from jax import lax
from jax.experimental import pallas as pl
from jax.experimental.pallas import tpu as pltpu
from jax.experimental.pallas import tpu_sc as plsc

E = 320000
D = 128
V = 100
NC = 2
NS = 16
NW = NC * NS
PER_W = E // NW
C = 80
G = C // 16
NCHUNK = PER_W // C
U = 8

_mesh = plsc.VectorSubcoreMesh(core_axis_name="c", subcore_axis_name="s")


@functools.partial(
    pl.kernel,
    out_type=jax.ShapeDtypeStruct((E * D,), jnp.float32),
    mesh=_mesh,
    compiler_params=pltpu.CompilerParams(needs_layout_passes=False),
    scratch_types=[
        pltpu.VMEM((PER_W,), jnp.int32),
        pltpu.VMEM((PER_W,), jnp.int32),
        pltpu.VMEM((PER_W,), jnp.int32),
        pltpu.VMEM((V * D,), jnp.float32),
        pltpu.VMEM((V * D,), jnp.float32),
        pltpu.VMEM((V * D,), jnp.float32),
        pltpu.VMEM((C * D,), jnp.float32),
    ],
)
def _bond_sum(i0_hbm, i1_hbm, i2_hbm, t0_hbm, t1_hbm, t2_hbm, out_hbm,
              i0_v, i1_v, i2_v, t0_v, t1_v, t2_v, ob):
    wid = lax.axis_index("s") * NC + lax.axis_index("c")
    base = wid * PER_W
    pltpu.sync_copy(i0_hbm.at[pl.ds(base, PER_W)], i0_v)
    pltpu.sync_copy(i1_hbm.at[pl.ds(base, PER_W)], i1_v)
    pltpu.sync_copy(i2_hbm.at[pl.ds(base, PER_W)], i2_v)
    pltpu.sync_copy(t0_hbm, t0_v)
    pltpu.sync_copy(t1_hbm, t1_v)
    pltpu.sync_copy(t2_hbm, t2_v)

    lane = lax.iota(jnp.int32, 16)

    def chunk(c, carry):
        off = c * C
        for g in range(G):
            s = off + g * 16
            a0 = i0_v[pl.ds(s, 16)] * D
            a1 = i1_v[pl.ds(s, 16)] * D
            a2 = i2_v[pl.ds(s, 16)] * D
            o0 = (lane + (g * 16)) * D

            def jbody(jj, cr, _g=g):
                a0c, a1c, a2c, oc = cr
                for u in range(U):
                    val = (plsc.load_gather(t0_v, [a0c + u])
                           + plsc.load_gather(t1_v, [a1c + u])
                           + plsc.load_gather(t2_v, [a2c + u]))
                    plsc.store_scatter(ob, [oc + u], val)
                return (a0c + U, a1c + U, a2c + U, oc + U)

            lax.fori_loop(0, D // U, jbody, (a0, a1, a2, o0))
        pltpu.sync_copy(ob, out_hbm.at[pl.ds((base + off) * D, C * D)])
        return carry

    lax.fori_loop(0, NCHUNK, chunk, 0)


def kernel(edge_attr, emb0, emb1, emb2):
    ea = edge_attr.astype(jnp.int32)
    out = _bond_sum(
        ea[:, 0], ea[:, 1], ea[:, 2],
        emb0.reshape(-1), emb1.reshape(-1), emb2.reshape(-1),
    )
    return out.reshape(E, D)


# speedup vs baseline: 3.5213x; 3.5213x over previous
import functools

import jax
import jax.numpy as jnp
from jax import lax
from jax.experimental import pallas as pl
from jax.experimental.pallas import tpu as pltpu
from jax.experimental.pallas import tpu_sc as plsc

E = 320000
D = 128
V = 100
NC = 2
NS = 16
NW = NC * NS
PER_W = E // NW
C = 80
NCHUNK = PER_W // C

_mesh = plsc.VectorSubcoreMesh(core_axis_name="c", subcore_axis_name="s")


@functools.partial(
    pl.kernel,
    out_type=jax.ShapeDtypeStruct((E, D), jnp.float32),
    mesh=_mesh,
    compiler_params=pltpu.CompilerParams(needs_layout_passes=False),
    scratch_types=[
        pltpu.VMEM((NCHUNK, C), jnp.int32),
        pltpu.VMEM((NCHUNK, C), jnp.int32),
        pltpu.VMEM((NCHUNK, C), jnp.int32),
        pltpu.VMEM((C, D), jnp.float32),
        pltpu.VMEM((C, D), jnp.float32),
        pltpu.VMEM((C, D), jnp.float32),
        pltpu.VMEM((C, D), jnp.float32),
        pltpu.VMEM((C, D), jnp.float32),
        pltpu.VMEM((C, D), jnp.float32),
        pltpu.SemaphoreType.DMA,
        pltpu.SemaphoreType.DMA,
        pltpu.SemaphoreType.DMA,
        pltpu.SemaphoreType.DMA,
    ],
)
def _bond_sum(i0_hbm, i1_hbm, i2_hbm, t0_hbm, t1_hbm, t2_hbm, out_hbm,
              i0_v, i1_v, i2_v, ob_a, ob_b, x_a, x_b, y_a, y_b,
              gs_a, gs_b, os_a, os_b):
    wid = lax.axis_index("s") * NC + lax.axis_index("c")
    row0 = wid * PER_W
    pltpu.sync_copy(i0_hbm.at[wid], i0_v)
    pltpu.sync_copy(i1_hbm.at[wid], i1_v)
    pltpu.sync_copy(i2_hbm.at[wid], i2_v)

    obs = (ob_a, ob_b)
    xs = (x_a, x_b)
    ys = (y_a, y_b)
    gs = (gs_a, gs_b)
    os_ = (os_a, os_b)

    def issue_gathers(c, b):
        pltpu.async_copy(t0_hbm.at[i0_v.at[c]], obs[b], gs[b])
        pltpu.async_copy(t1_hbm.at[i1_v.at[c]], xs[b], gs[b])
        pltpu.async_copy(t2_hbm.at[i2_v.at[c]], ys[b], gs[b])

    def wait_gathers(c, b):
        pltpu.make_async_copy(t0_hbm.at[i0_v.at[c]], obs[b], gs[b]).wait()
        pltpu.make_async_copy(t1_hbm.at[i1_v.at[c]], xs[b], gs[b]).wait()
        pltpu.make_async_copy(t2_hbm.at[i2_v.at[c]], ys[b], gs[b]).wait()

    def issue_out(c, b):
        r = pl.multiple_of(row0 + c * C, 16)
        pltpu.async_copy(obs[b], out_hbm.at[pl.ds(r, C)], os_[b])

    def wait_out(c, b):
        r = pl.multiple_of(row0 + c * C, 16)
        pltpu.make_async_copy(obs[b], out_hbm.at[pl.ds(r, C)],
                              os_[b]).wait()

    def compute(b):
        o, x, y = obs[b], xs[b], ys[b]

        @plsc.parallel_loop(0, C, unroll=2)
        def _(r):
            for j in range(D // 16):
                sl = pl.ds(j * 16, 16)
                plsc.addupdate(o.at[r, sl], x[r, sl] + y[r, sl])

    issue_gathers(0, 0)
    wait_gathers(0, 0)
    issue_gathers(1, 1)
    compute(0)
    issue_out(0, 0)

    def pair(i, carry):
        for (off, b) in ((1, 1), (2, 0)):
            c = 2 * i + off
            wait_gathers(c, b)
            wait_out(c - 1, 1 - b)
            issue_gathers(c + 1, 1 - b)
            compute(b)
            issue_out(c, b)
        return carry

    lax.fori_loop(0, (NCHUNK - 3) // 2, pair, 0)

    c = NCHUNK - 2
    wait_gathers(c, 1)
    wait_out(c - 1, 0)
    issue_gathers(c + 1, 0)
    compute(1)
    issue_out(c, 1)

    c = NCHUNK - 1
    wait_gathers(c, 0)
    wait_out(c - 1, 1)
    compute(0)
    issue_out(c, 0)
    wait_out(c, 0)


def kernel(edge_attr, emb0, emb1, emb2):
    ea = edge_attr.astype(jnp.int32)
    i0 = ea[:, 0].reshape(NW, NCHUNK, C)
    i1 = ea[:, 1].reshape(NW, NCHUNK, C)
    i2 = ea[:, 2].reshape(NW, NCHUNK, C)
    return _bond_sum(i0, i1, i2, emb0, emb1, emb2)


# speedup vs baseline: 12.8173x; 3.6399x over previous
import functools

import jax
import jax.numpy as jnp
from jax import lax
from jax.experimental import pallas as pl
from jax.experimental.pallas import tpu as pltpu
from jax.experimental.pallas import tpu_sc as plsc

E = 320000
D = 128
V = 100
NC = 2
NS = 16
NW = NC * NS
PER_W = E // NW
C = 80
NCHUNK = PER_W // C

_mesh = plsc.VectorSubcoreMesh(core_axis_name="c", subcore_axis_name="s")


@functools.partial(
    pl.kernel,
    out_type=jax.ShapeDtypeStruct((E, D), jnp.float32),
    mesh=_mesh,
    compiler_params=pltpu.CompilerParams(needs_layout_passes=False),
    scratch_types=[
        pltpu.VMEM((NCHUNK, C), jnp.int32),
        pltpu.VMEM((NCHUNK, C), jnp.int32),
        pltpu.VMEM((NCHUNK, C), jnp.int32),
        pltpu.VMEM((C, D), jnp.float32),
        pltpu.VMEM((C, D), jnp.float32),
        pltpu.VMEM((C, D), jnp.float32),
        pltpu.VMEM((C, D), jnp.float32),
        pltpu.VMEM((C, D), jnp.float32),
        pltpu.VMEM((C, D), jnp.float32),
        pltpu.VMEM_SHARED((V, D), jnp.float32),
        pltpu.VMEM_SHARED((V, D), jnp.float32),
        pltpu.VMEM_SHARED((V, D), jnp.float32),
        pltpu.SemaphoreType.DMA,
        pltpu.SemaphoreType.DMA,
        pltpu.SemaphoreType.DMA,
        pltpu.SemaphoreType.DMA,
    ],
)
def _bond_sum(i0_hbm, i1_hbm, i2_hbm, t0_hbm, t1_hbm, t2_hbm, out_hbm,
              i0_v, i1_v, i2_v, ob_a, ob_b, x_a, x_b, y_a, y_b,
              t0_s, t1_s, t2_s,
              gs_a, gs_b, os_a, os_b):
    wid = lax.axis_index("s") * NC + lax.axis_index("c")
    row0 = wid * PER_W
    sid = lax.axis_index("s")

    @pl.when(sid == 0)
    def _stage_tables():
        pltpu.sync_copy(t0_hbm, t0_s)
        pltpu.sync_copy(t1_hbm, t1_s)
        pltpu.sync_copy(t2_hbm, t2_s)

    pltpu.sync_copy(i0_hbm.at[wid], i0_v)
    pltpu.sync_copy(i1_hbm.at[wid], i1_v)
    pltpu.sync_copy(i2_hbm.at[wid], i2_v)
    plsc.subcore_barrier()

    obs = (ob_a, ob_b)
    xs = (x_a, x_b)
    ys = (y_a, y_b)
    gs = (gs_a, gs_b)
    os_ = (os_a, os_b)

    def issue_gathers(c, b):
        pltpu.async_copy(t0_s.at[i0_v.at[c]], obs[b], gs[b])
        pltpu.async_copy(t1_s.at[i1_v.at[c]], xs[b], gs[b])
        pltpu.async_copy(t2_s.at[i2_v.at[c]], ys[b], gs[b])

    def wait_gathers(c, b):
        pltpu.make_async_copy(t0_s.at[i0_v.at[c]], obs[b], gs[b]).wait()
        pltpu.make_async_copy(t1_s.at[i1_v.at[c]], xs[b], gs[b]).wait()
        pltpu.make_async_copy(t2_s.at[i2_v.at[c]], ys[b], gs[b]).wait()

    def issue_out(c, b):
        r = pl.multiple_of(row0 + c * C, 16)
        pltpu.async_copy(obs[b], out_hbm.at[pl.ds(r, C)], os_[b])

    def wait_out(c, b):
        r = pl.multiple_of(row0 + c * C, 16)
        pltpu.make_async_copy(obs[b], out_hbm.at[pl.ds(r, C)],
                              os_[b]).wait()

    def compute(b):
        o, x, y = obs[b], xs[b], ys[b]

        @plsc.parallel_loop(0, C, unroll=2)
        def _(r):
            for j in range(D // 16):
                sl = pl.ds(j * 16, 16)
                plsc.addupdate(o.at[r, sl], x[r, sl] + y[r, sl])

    issue_gathers(0, 0)
    wait_gathers(0, 0)
    issue_gathers(1, 1)
    compute(0)
    issue_out(0, 0)

    def pair(i, carry):
        for (off, b) in ((1, 1), (2, 0)):
            c = 2 * i + off
            wait_gathers(c, b)
            wait_out(c - 1, 1 - b)
            issue_gathers(c + 1, 1 - b)
            compute(b)
            issue_out(c, b)
        return carry

    lax.fori_loop(0, (NCHUNK - 3) // 2, pair, 0)

    c = NCHUNK - 2
    wait_gathers(c, 1)
    wait_out(c - 1, 0)
    issue_gathers(c + 1, 0)
    compute(1)
    issue_out(c, 1)

    c = NCHUNK - 1
    wait_gathers(c, 0)
    wait_out(c - 1, 1)
    compute(0)
    issue_out(c, 0)
    wait_out(c, 0)


def kernel(edge_attr, emb0, emb1, emb2):
    ea = edge_attr.astype(jnp.int32)
    i0 = ea[:, 0].reshape(NW, NCHUNK, C)
    i1 = ea[:, 1].reshape(NW, NCHUNK, C)
    i2 = ea[:, 2].reshape(NW, NCHUNK, C)
    return _bond_sum(i0, i1, i2, emb0, emb1, emb2)


# speedup vs baseline: 12.8385x; 1.0017x over previous
import jax
import jax.numpy as jnp
from jax import lax
from jax.experimental import pallas as pl
from jax.experimental.pallas import tpu as pltpu
from jax.experimental.pallas import tpu_sc as plsc

E = 320000
D = 128
V = 100
NC = 2
NS = 16
NW = NC * NS
PER_W = E // NW
C = 80
NCHUNK = PER_W // C

_mesh = plsc.VectorSubcoreMesh(core_axis_name="c", subcore_axis_name="s")


@pl.kernel(
    out_type=jax.ShapeDtypeStruct((E, D), jnp.float32),
    mesh=_mesh,
    compiler_params=pltpu.CompilerParams(needs_layout_passes=False),
    scratch_types=[
        pltpu.VMEM((NCHUNK, C), jnp.int32),
        pltpu.VMEM((NCHUNK, C), jnp.int32),
        pltpu.VMEM((NCHUNK, C), jnp.int32),
        pltpu.VMEM((C, D), jnp.float32),
        pltpu.VMEM((C, D), jnp.float32),
        pltpu.VMEM((C, D), jnp.float32),
        pltpu.VMEM((C, D), jnp.float32),
        pltpu.VMEM((C, D), jnp.float32),
        pltpu.VMEM((C, D), jnp.float32),
        pltpu.VMEM_SHARED((V, D), jnp.float32),
        pltpu.VMEM_SHARED((V, D), jnp.float32),
        pltpu.VMEM_SHARED((V, D), jnp.float32),
        pltpu.SemaphoreType.DMA,
        pltpu.SemaphoreType.DMA,
        pltpu.SemaphoreType.DMA,
        pltpu.SemaphoreType.DMA,
    ],
)
def _bond_sum(i0_hbm, i1_hbm, i2_hbm, t0_hbm, t1_hbm, t2_hbm, out_hbm,
              i0_v, i1_v, i2_v, ob_a, ob_b, x_a, x_b, y_a, y_b,
              t0_s, t1_s, t2_s,
              gs_a, gs_b, os_a, os_b):
    wid = lax.axis_index("s") * NC + lax.axis_index("c")
    row0 = wid * PER_W
    sid = lax.axis_index("s")

    @pl.when(sid == 0)
    def _stage_tables():
        pltpu.sync_copy(t0_hbm, t0_s)
        pltpu.sync_copy(t1_hbm, t1_s)
        pltpu.sync_copy(t2_hbm, t2_s)

    pltpu.sync_copy(i0_hbm.at[wid], i0_v)
    pltpu.sync_copy(i1_hbm.at[wid], i1_v)
    pltpu.sync_copy(i2_hbm.at[wid], i2_v)
    plsc.subcore_barrier()

    obs = (ob_a, ob_b)
    xs = (x_a, x_b)
    ys = (y_a, y_b)
    gs = (gs_a, gs_b)
    os_ = (os_a, os_b)

    def issue_gathers(c, b):
        pltpu.async_copy(t0_s.at[i0_v.at[c]], obs[b], gs[b])
        pltpu.async_copy(t1_s.at[i1_v.at[c]], xs[b], gs[b])
        pltpu.async_copy(t2_s.at[i2_v.at[c]], ys[b], gs[b])

    def wait_gathers(c, b):
        pltpu.make_async_copy(t0_s.at[i0_v.at[c]], obs[b], gs[b]).wait()
        pltpu.make_async_copy(t1_s.at[i1_v.at[c]], xs[b], gs[b]).wait()
        pltpu.make_async_copy(t2_s.at[i2_v.at[c]], ys[b], gs[b]).wait()

    def issue_out(c, b):
        r = pl.multiple_of(row0 + c * C, 16)
        pltpu.async_copy(obs[b], out_hbm.at[pl.ds(r, C)], os_[b])

    def wait_out(c, b):
        r = pl.multiple_of(row0 + c * C, 16)
        pltpu.make_async_copy(obs[b], out_hbm.at[pl.ds(r, C)],
                              os_[b]).wait()

    def compute(b):
        o, x, y = obs[b], xs[b], ys[b]

        @plsc.parallel_loop(0, C, unroll=2)
        def _(r):
            for j in range(D // 16):
                sl = pl.ds(j * 16, 16)
                plsc.addupdate(o.at[r, sl], x[r, sl] + y[r, sl])

    issue_gathers(0, 0)
    wait_gathers(0, 0)
    issue_gathers(1, 1)
    compute(0)
    issue_out(0, 0)

    def pair(i, carry):
        for (off, b) in ((1, 1), (2, 0)):
            c = 2 * i + off
            wait_gathers(c, b)
            wait_out(c - 1, 1 - b)
            issue_gathers(c + 1, 1 - b)
            compute(b)
            issue_out(c, b)
        return carry

    lax.fori_loop(0, (NCHUNK - 3) // 2, pair, 0)

    c = NCHUNK - 2
    wait_gathers(c, 1)
    wait_out(c - 1, 0)
    issue_gathers(c + 1, 0)
    compute(1)
    issue_out(c, 1)

    c = NCHUNK - 1
    wait_gathers(c, 0)
    wait_out(c - 1, 1)
    compute(0)
    issue_out(c, 0)
    wait_out(c, 0)


def kernel(edge_attr, emb0, emb1, emb2):
    ea = edge_attr.astype(jnp.int32)
    i0 = ea[:, 0].reshape(NW, NCHUNK, C)
    i1 = ea[:, 1].reshape(NW, NCHUNK, C)
    i2 = ea[:, 2].reshape(NW, NCHUNK, C)
    return _bond_sum(i0, i1, i2, emb0, emb1, emb2)
